# baseline (speedup 1.0000x reference)
import functools

import jax
import jax.numpy as jnp
from jax import lax
from jax.experimental import pallas as pl
from jax.experimental.pallas import tpu as pltpu
from jax.experimental.pallas import tpu_sc as plsc

_N = 10000
_D = 128
_H = 128
_E = 320000

_NC = 2
_NS = 16
_NW = _NC * _NS

_CHUNK = 128
_NCHUNK = 80
_NB = 4
_EPT = _CHUNK * _NCHUNK
_EPAD = _EPT * _NW
_NPAD = 10240
_RPT = _NPAD // _NW
_RPS = _NPAD // _NS

_ROWBLK = 512
_GRID = _NPAD // _ROWBLK

@functools.cache
def _mesh():
    return plsc.VectorSubcoreMesh(
        core_axis_name="c", subcore_axis_name="s",
        num_cores=_NC, num_subcores=_NS)


def _deg_body(dst_hbm, out_hbm, idx_d, ones_v, zrow, deg_sh, sem):
    c = lax.axis_index("c")
    s = lax.axis_index("s")
    wid = c * _NS + s

    one16 = jnp.ones((16,), jnp.float32)
    zero16 = jnp.zeros((16,), jnp.float32)
    for j in range(_CHUNK // 16):
        ones_v[pl.ds(j * 16, 16)] = one16

    def zfill(i, _):
        zrow[pl.ds(i * 16, 16)] = zero16
        return 0
    lax.fori_loop(0, _RPS // 16, zfill, 0)
    pltpu.sync_copy(zrow, deg_sh.at[pl.ds(s * _RPS, _RPS)])
    plsc.subcore_barrier()

    base = wid * _EPT

    def body(j, _):
        off = pl.multiple_of(base + j * _CHUNK, 8)
        pltpu.sync_copy(dst_hbm.at[pl.ds(off, _CHUNK)], idx_d)
        pltpu.sync_copy(ones_v, deg_sh.at[idx_d], add=True)
        return 0
    lax.fori_loop(0, _NCHUNK, body, 0)

    plsc.subcore_barrier()
    pltpu.sync_copy(deg_sh.at[pl.ds(s * _RPS, _RPS)], zrow)
    pltpu.sync_copy(zrow, out_hbm.at[c, pl.ds(s * _RPS, _RPS)])


@functools.cache
def _deg_call():
    return pl.kernel(
        _deg_body,
        out_type=jax.ShapeDtypeStruct((_NC, _NPAD), jnp.float32),
        mesh=_mesh(),
        scratch_types=[
            pltpu.VMEM((_CHUNK,), jnp.int32),
            pltpu.VMEM((_CHUNK,), jnp.float32),
            pltpu.VMEM((_RPS,), jnp.float32),
            pltpu.VMEM_SHARED((_NPAD,), jnp.float32),
            pltpu.SemaphoreType.DMA,
        ],
    )


def _msg_body(g_hbm, src_hbm, dst_hbm, out_hbm, idx_s, idx_d, rows, zbuf,
              acc_sh, sem):
    c = lax.axis_index("c")
    s = lax.axis_index("s")
    wid = c * _NS + s

    zero16 = jnp.zeros((16,), jnp.float32)

    def zfill(i, _):
        for j in range(_H // 16):
            zbuf[i, pl.ds(j * 16, 16)] = zero16
        return 0
    lax.fori_loop(0, 64, zfill, 0)
    for k in range(_RPS // 64):
        pltpu.sync_copy(zbuf, acc_sh.at[pl.ds(s * _RPS + k * 64, 64)])
    plsc.subcore_barrier()

    base = wid * _EPT

    def body(j, _):
        off = pl.multiple_of(base + j * _CHUNK, 8)
        pltpu.sync_copy(src_hbm.at[pl.ds(off, _CHUNK)], idx_s)
        pltpu.sync_copy(dst_hbm.at[pl.ds(off, _CHUNK)], idx_d)
        pltpu.async_copy(g_hbm.at[idx_s], rows, sem).wait()
        pltpu.sync_copy(rows, acc_sh.at[idx_d], add=True)
        return 0
    lax.fori_loop(0, _NCHUNK, body, 0)

    plsc.subcore_barrier()
    r0 = s * _RPS
    for k in range(_RPS // 64):
        pltpu.sync_copy(acc_sh.at[pl.ds(r0 + k * 64, 64)], zbuf)
        pltpu.sync_copy(zbuf, out_hbm.at[c, pl.ds(r0 + k * 64, 64)])


@functools.cache
def _msg_call():
    return pl.kernel(
        _msg_body,
        out_type=jax.ShapeDtypeStruct((_NC, _NPAD, _H), jnp.float32),
        mesh=_mesh(),
        scratch_types=[
            pltpu.VMEM((_CHUNK,), jnp.int32),
            pltpu.VMEM((_CHUNK,), jnp.int32),
            pltpu.VMEM((_CHUNK, _H), jnp.float32),
            pltpu.VMEM((64, _H), jnp.float32),
            pltpu.VMEM_SHARED((_NPAD, _H), jnp.float32),
            pltpu.SemaphoreType.DMA,
        ],
    )


def _rowspec(w):
    return pl.BlockSpec((_ROWBLK, w), lambda i: (i, 0))


def _fullspec(r, c):
    return pl.BlockSpec((r, c), lambda i: (0, 0))


def _dot(a, b):
    return jnp.dot(a, b, preferred_element_type=jnp.float32)


def _enc_body(x, w1, b1, w2, b2, o):
    h = jnp.maximum(_dot(x[...], w1[...]) + b1[...], 0.0)
    o[...] = _dot(h, w2[...]) + b2[...]


def _enc_call(x, w1, b1, w2, b2):
    return pl.pallas_call(
        _enc_body,
        grid=(_GRID,),
        in_specs=[_rowspec(_D), _fullspec(_D, _H), _fullspec(1, _H),
                  _fullspec(_H, _H), _fullspec(1, _H)],
        out_specs=_rowspec(_H),
        out_shape=jax.ShapeDtypeStruct((_NPAD, _H), jnp.float32),
    )(x, w1, b1, w2, b2)


def _pre_body(h, d0, d1, w, g, dinv):
    deg = d0[...] + d1[...] + 1.0
    dv = lax.rsqrt(deg)
    dv = dv * (1.5 - 0.5 * deg * dv * dv)
    dinv[...] = dv
    g[...] = dv * _dot(h[...], w[...])


def _pre_call(h, d0, d1, w):
    return pl.pallas_call(
        _pre_body,
        grid=(_GRID,),
        in_specs=[_rowspec(_H), _rowspec(1), _rowspec(1), _fullspec(_H, _H)],
        out_specs=(_rowspec(_H), _rowspec(1)),
        out_shape=(jax.ShapeDtypeStruct((_NPAD, _H), jnp.float32),
                   jax.ShapeDtypeStruct((_NPAD, 1), jnp.float32)),
    )(h, d0, d1, w)


def _mid_body(a0, a1, g, dinv, bp, wn, o):
    dv = dinv[...]
    h = jnp.maximum(dv * (a0[...] + a1[...] + g[...]) + bp[...], 0.0)
    o[...] = dv * _dot(h, wn[...])


def _mid_call(a0, a1, g, dinv, bp, wn):
    return pl.pallas_call(
        _mid_body,
        grid=(_GRID,),
        in_specs=[_rowspec(_H), _rowspec(_H), _rowspec(_H), _rowspec(1),
                  _fullspec(1, _H), _fullspec(_H, _H)],
        out_specs=_rowspec(_H),
        out_shape=jax.ShapeDtypeStruct((_NPAD, _H), jnp.float32),
    )(a0, a1, g, dinv, bp, wn)


def _dec_body(a0, a1, g, dinv, bc, w1, b1, w2, b2, o):
    h = jnp.maximum(dinv[...] * (a0[...] + a1[...] + g[...]) + bc[...], 0.0)
    t = jnp.maximum(_dot(h, w1[...]) + b1[...], 0.0)
    o[...] = jax.nn.sigmoid(_dot(t, w2[...]) + b2[...])


def _dec_call(a0, a1, g, dinv, bc, w1, b1, w2, b2):
    return pl.pallas_call(
        _dec_body,
        grid=(_GRID,),
        in_specs=[_rowspec(_H), _rowspec(_H), _rowspec(_H), _rowspec(1),
                  _fullspec(1, _H), _fullspec(_H, _H), _fullspec(1, _H),
                  _fullspec(_H, _D), _fullspec(1, _D)],
        out_specs=_rowspec(_D),
        out_shape=jax.ShapeDtypeStruct((_NPAD, _D), jnp.float32),
    )(a0, a1, g, dinv, bc, w1, b1, w2, b2)


def kernel(x, edge_index, W_e1, b_e1, W_e2, b_e2, W_c1, b_c1, W_c2, b_c2,
           W_c3, b_c3, W_d1, b_d1, W_d2, b_d2):
    src, dst = edge_index[0], edge_index[1]
    pad = jnp.full((_EPAD - _E,), _N, jnp.int32)
    srcp = jnp.concatenate([src, pad])
    dstp = jnp.concatenate([dst, pad])
    xp = jnp.pad(x, ((0, _NPAD - _N), (0, 0)))

    degp = _deg_call()(dstp)
    d0 = degp[0].reshape(_NPAD, 1)
    d1 = degp[1].reshape(_NPAD, 1)

    h = _enc_call(xp, W_e1, b_e1.reshape(1, _H), W_e2, b_e2.reshape(1, _H))
    g1, dinv = _pre_call(h, d0, d1, W_c1)
    msg = _msg_call()
    acc = msg(g1, srcp, dstp)
    g2 = _mid_call(acc[0], acc[1], g1, dinv, b_c1.reshape(1, _H), W_c2)
    acc = msg(g2, srcp, dstp)
    g3 = _mid_call(acc[0], acc[1], g2, dinv, b_c2.reshape(1, _H), W_c3)
    acc = msg(g3, srcp, dstp)
    out = _dec_call(acc[0], acc[1], g3, dinv, b_c3.reshape(1, _H),
                    W_d1, b_d1.reshape(1, _H), W_d2, b_d2.reshape(1, _D))
    return out[:_N]


# speedup vs baseline: 2.1003x; 2.1003x over previous
import functools

import jax
import jax.numpy as jnp
from jax import lax
from jax.experimental import pallas as pl
from jax.experimental.pallas import tpu as pltpu
from jax.experimental.pallas import tpu_sc as plsc

_N = 10000
_D = 128
_H = 128
_E = 320000

_NC = 2
_NS = 16
_NW = _NC * _NS

_CHUNK = 128
_NCHUNK = 80
_NB = 4
_EPT = _CHUNK * _NCHUNK
_EPAD = _EPT * _NW
_NPAD = 10240
_RPT = _NPAD // _NW
_RPS = _NPAD // _NS

_ROWBLK = 512
_GRID = _NPAD // _ROWBLK

@functools.cache
def _mesh():
    return plsc.VectorSubcoreMesh(
        core_axis_name="c", subcore_axis_name="s",
        num_cores=_NC, num_subcores=_NS)


def _deg_body(dst_hbm, out_hbm, idx_d, ones_v, zrow, deg_sh, sem):
    c = lax.axis_index("c")
    s = lax.axis_index("s")
    wid = c * _NS + s

    one16 = jnp.ones((16,), jnp.float32)
    zero16 = jnp.zeros((16,), jnp.float32)
    for j in range(_CHUNK // 16):
        ones_v[pl.ds(j * 16, 16)] = one16

    def zfill(i, _):
        zrow[pl.ds(i * 16, 16)] = zero16
        return 0
    lax.fori_loop(0, _RPS // 16, zfill, 0)
    pltpu.sync_copy(zrow, deg_sh.at[pl.ds(s * _RPS, _RPS)])
    plsc.subcore_barrier()

    base = wid * _EPT

    def body(j, _):
        off = pl.multiple_of(base + j * _CHUNK, 8)
        pltpu.sync_copy(dst_hbm.at[pl.ds(off, _CHUNK)], idx_d)
        pltpu.sync_copy(ones_v, deg_sh.at[idx_d], add=True)
        return 0
    lax.fori_loop(0, _NCHUNK, body, 0)

    plsc.subcore_barrier()
    pltpu.sync_copy(deg_sh.at[pl.ds(s * _RPS, _RPS)], zrow)
    pltpu.sync_copy(zrow, out_hbm.at[c, pl.ds(s * _RPS, _RPS)])


@functools.cache
def _deg_call():
    return pl.kernel(
        _deg_body,
        out_type=jax.ShapeDtypeStruct((_NC, _NPAD), jnp.float32),
        mesh=_mesh(),
        scratch_types=[
            pltpu.VMEM((_CHUNK,), jnp.int32),
            pltpu.VMEM((_CHUNK,), jnp.float32),
            pltpu.VMEM((_RPS,), jnp.float32),
            pltpu.VMEM_SHARED((_NPAD,), jnp.float32),
            pltpu.SemaphoreType.DMA,
        ],
    )


def _msg_body(g_hbm, src_hbm, dst_hbm, out_hbm, idx_s, idx_d, rows, zbuf,
              acc_sh, sem):
    c = lax.axis_index("c")
    s = lax.axis_index("s")
    wid = c * _NS + s

    zero16 = jnp.zeros((16,), jnp.float32)

    def zfill(i, _):
        for j in range(_H // 16):
            zbuf[i, pl.ds(j * 16, 16)] = zero16
        return 0
    lax.fori_loop(0, 64, zfill, 0)
    for k in range(_RPS // 64):
        pltpu.sync_copy(zbuf, acc_sh.at[pl.ds(s * _RPS + k * 64, 64)])
    plsc.subcore_barrier()

    base = wid * _EPT

    def body(j, _):
        off = pl.multiple_of(base + j * _CHUNK, 8)
        pltpu.sync_copy(src_hbm.at[pl.ds(off, _CHUNK)], idx_s)
        pltpu.sync_copy(dst_hbm.at[pl.ds(off, _CHUNK)], idx_d)
        pltpu.async_copy(g_hbm.at[idx_s], rows, sem).wait()
        pltpu.sync_copy(rows, acc_sh.at[idx_d], add=True)
        return 0
    lax.fori_loop(0, _NCHUNK, body, 0)

    plsc.subcore_barrier()
    r0 = s * _RPS
    for k in range(_RPS // 64):
        pltpu.sync_copy(acc_sh.at[pl.ds(r0 + k * 64, 64)], zbuf)
        pltpu.sync_copy(zbuf, out_hbm.at[c, pl.ds(r0 + k * 64, 64)])


@functools.cache
def _msg_call():
    return pl.kernel(
        _msg_body,
        out_type=jax.ShapeDtypeStruct((_NC, _NPAD, _H), jnp.float32),
        mesh=_mesh(),
        scratch_types=[
            pltpu.VMEM((_CHUNK,), jnp.int32),
            pltpu.VMEM((_CHUNK,), jnp.int32),
            pltpu.VMEM((_CHUNK, _H), jnp.float32),
            pltpu.VMEM((64, _H), jnp.float32),
            pltpu.VMEM_SHARED((_NPAD, _H), jnp.float32),
            pltpu.SemaphoreType.DMA,
        ],
    )


def _rowspec(w):
    return pl.BlockSpec((_ROWBLK, w), lambda i: (i, 0))


def _fullspec(r, c):
    return pl.BlockSpec((r, c), lambda i: (0, 0))


def _dot(a, b):
    return jnp.dot(a, b, preferred_element_type=jnp.float32)


def _enc_body(x, w1, b1, w2, b2, o):
    h = jnp.maximum(_dot(x[...], w1[...]) + b1[...], 0.0)
    o[...] = _dot(h, w2[...]) + b2[...]


def _enc_call(x, w1, b1, w2, b2):
    return pl.pallas_call(
        _enc_body,
        grid=(_GRID,),
        in_specs=[_rowspec(_D), _fullspec(_D, _H), _fullspec(1, _H),
                  _fullspec(_H, _H), _fullspec(1, _H)],
        out_specs=_rowspec(_H),
        out_shape=jax.ShapeDtypeStruct((_NPAD, _H), jnp.float32),
    )(x, w1, b1, w2, b2)


def _pre_body(h, d0, d1, w, g, dinv):
    deg = d0[...] + d1[...] + 1.0
    dv = lax.rsqrt(deg)
    dv = dv * (1.5 - 0.5 * deg * dv * dv)
    dinv[...] = dv
    g[...] = dv * _dot(h[...], w[...])


def _pre_call(h, d0, d1, w):
    return pl.pallas_call(
        _pre_body,
        grid=(_GRID,),
        in_specs=[_rowspec(_H), _rowspec(1), _rowspec(1), _fullspec(_H, _H)],
        out_specs=(_rowspec(_H), _rowspec(1)),
        out_shape=(jax.ShapeDtypeStruct((_NPAD, _H), jnp.float32),
                   jax.ShapeDtypeStruct((_NPAD, 1), jnp.float32)),
    )(h, d0, d1, w)


def _mid_body(a0, a1, g, dinv, bp, wn, o):
    dv = dinv[...]
    h = jnp.maximum(dv * (a0[...] + a1[...] + g[...]) + bp[...], 0.0)
    o[...] = dv * _dot(h, wn[...])


def _mid_call(a0, a1, g, dinv, bp, wn):
    return pl.pallas_call(
        _mid_body,
        grid=(_GRID,),
        in_specs=[_rowspec(_H), _rowspec(_H), _rowspec(_H), _rowspec(1),
                  _fullspec(1, _H), _fullspec(_H, _H)],
        out_specs=_rowspec(_H),
        out_shape=jax.ShapeDtypeStruct((_NPAD, _H), jnp.float32),
    )(a0, a1, g, dinv, bp, wn)


def _dec_body(a0, a1, g, dinv, bc, w1, b1, w2, b2, o):
    h = jnp.maximum(dinv[...] * (a0[...] + a1[...] + g[...]) + bc[...], 0.0)
    t = jnp.maximum(_dot(h, w1[...]) + b1[...], 0.0)
    o[...] = jax.nn.sigmoid(_dot(t, w2[...]) + b2[...])


def _dec_call(a0, a1, g, dinv, bc, w1, b1, w2, b2):
    return pl.pallas_call(
        _dec_body,
        grid=(_GRID,),
        in_specs=[_rowspec(_H), _rowspec(_H), _rowspec(_H), _rowspec(1),
                  _fullspec(1, _H), _fullspec(_H, _H), _fullspec(1, _H),
                  _fullspec(_H, _D), _fullspec(1, _D)],
        out_specs=_rowspec(_D),
        out_shape=jax.ShapeDtypeStruct((_NPAD, _D), jnp.float32),
    )(a0, a1, g, dinv, bc, w1, b1, w2, b2)


def kernel(x, edge_index, W_e1, b_e1, W_e2, b_e2, W_c1, b_c1, W_c2, b_c2,
           W_c3, b_c3, W_d1, b_d1, W_d2, b_d2):
    src, dst = edge_index[0], edge_index[1]
    pad = _N + (jnp.arange(_EPAD - _E, dtype=jnp.int32) % (_NPAD - _N))
    srcp = jnp.concatenate([src, pad])
    dstp = jnp.concatenate([dst, pad])
    xp = jnp.pad(x, ((0, _NPAD - _N), (0, 0)))

    degp = _deg_call()(dstp)
    d0 = degp[0].reshape(_NPAD, 1)
    d1 = degp[1].reshape(_NPAD, 1)

    h = _enc_call(xp, W_e1, b_e1.reshape(1, _H), W_e2, b_e2.reshape(1, _H))
    g1, dinv = _pre_call(h, d0, d1, W_c1)
    msg = _msg_call()
    acc = msg(g1, srcp, dstp)
    g2 = _mid_call(acc[0], acc[1], g1, dinv, b_c1.reshape(1, _H), W_c2)
    acc = msg(g2, srcp, dstp)
    g3 = _mid_call(acc[0], acc[1], g2, dinv, b_c2.reshape(1, _H), W_c3)
    acc = msg(g3, srcp, dstp)
    out = _dec_call(acc[0], acc[1], g3, dinv, b_c3.reshape(1, _H),
                    W_d1, b_d1.reshape(1, _H), W_d2, b_d2.reshape(1, _D))
    return out[:_N]


# speedup vs baseline: 3.8679x; 1.8416x over previous
import functools

import jax
import jax.numpy as jnp
from jax import lax
from jax.experimental import pallas as pl
from jax.experimental.pallas import tpu as pltpu
from jax.experimental.pallas import tpu_sc as plsc

_N = 10000
_D = 128
_H = 128
_E = 320000

_NC = 2
_NS = 16
_NW = _NC * _NS

_CHUNK = 128
_NCHUNK = 80
_NB = 4
_EPT = _CHUNK * _NCHUNK
_EPAD = _EPT * _NW
_NPAD = 10240
_RPT = _NPAD // _NW
_RPS = _NPAD // _NS

_ROWBLK = 512
_GRID = _NPAD // _ROWBLK

@functools.cache
def _mesh():
    return plsc.VectorSubcoreMesh(
        core_axis_name="c", subcore_axis_name="s",
        num_cores=_NC, num_subcores=_NS)


def _deg_body(dst_hbm, out_hbm, idx_d, ones_v, zrow, deg_sh, sem):
    c = lax.axis_index("c")
    s = lax.axis_index("s")
    wid = c * _NS + s

    one16 = jnp.ones((16,), jnp.float32)
    zero16 = jnp.zeros((16,), jnp.float32)
    for j in range(_CHUNK // 16):
        ones_v[pl.ds(j * 16, 16)] = one16

    def zfill(i, _):
        zrow[pl.ds(i * 16, 16)] = zero16
        return 0
    lax.fori_loop(0, _RPS // 16, zfill, 0)
    pltpu.sync_copy(zrow, deg_sh.at[pl.ds(s * _RPS, _RPS)])
    plsc.subcore_barrier()

    base = wid * _EPT

    def body(j, _):
        off = pl.multiple_of(base + j * _CHUNK, 8)
        pltpu.sync_copy(dst_hbm.at[pl.ds(off, _CHUNK)], idx_d)
        pltpu.sync_copy(ones_v, deg_sh.at[idx_d], add=True)
        return 0
    lax.fori_loop(0, _NCHUNK, body, 0)

    plsc.subcore_barrier()
    pltpu.sync_copy(deg_sh.at[pl.ds(s * _RPS, _RPS)], zrow)
    pltpu.sync_copy(zrow, out_hbm.at[c, pl.ds(s * _RPS, _RPS)])


@functools.cache
def _deg_call():
    return pl.kernel(
        _deg_body,
        out_type=jax.ShapeDtypeStruct((_NC, _NPAD), jnp.float32),
        mesh=_mesh(),
        scratch_types=[
            pltpu.VMEM((_CHUNK,), jnp.int32),
            pltpu.VMEM((_CHUNK,), jnp.float32),
            pltpu.VMEM((_RPS,), jnp.float32),
            pltpu.VMEM_SHARED((_NPAD,), jnp.float32),
            pltpu.SemaphoreType.DMA,
        ],
    )


def _msg_body(g_hbm, sd_hbm, out_hbm, idx, rows, zbuf, acc_sh,
              isem0, isem1, isem2, isem3, gsem0, gsem1):
    c = lax.axis_index("c")
    s = lax.axis_index("s")
    wid = c * _NS + s
    isems = (isem0, isem1, isem2, isem3)
    gsems = (gsem0, gsem1)

    c0 = wid * _NCHUNK
    for b in range(_NB):
        pltpu.async_copy(sd_hbm.at[c0 + b], idx.at[b], isems[b])

    zero16 = jnp.zeros((16,), jnp.float32)

    def zfill(i, _):
        for j in range(_H // 16):
            zbuf[i, pl.ds(j * 16, 16)] = zero16
        return 0
    lax.fori_loop(0, 64, zfill, 0)
    for k in range(_RPS // 64):
        pltpu.sync_copy(zbuf, acc_sh.at[pl.ds(s * _RPS + k * 64, 64)])
    plsc.subcore_barrier()

    for b in range(2):
        pltpu.make_async_copy(sd_hbm.at[0], idx.at[b], isems[b]).wait()
        pltpu.async_copy(g_hbm.at[idx.at[b, 0]], rows.at[b], gsems[b])

    def grp(jg, _):
        for b in range(_NB):
            j = jg * _NB + b
            rb = b % 2
            pltpu.make_async_copy(
                g_hbm.at[pl.ds(0, _CHUNK)], rows.at[rb], gsems[rb]).wait()
            pltpu.sync_copy(rows.at[rb], acc_sh.at[idx.at[b, 1]], add=True)

            @pl.when(j + _NB < _NCHUNK)
            def _():
                pltpu.async_copy(sd_hbm.at[c0 + j + _NB], idx.at[b], isems[b])

            @pl.when(j + 2 < _NCHUNK)
            def _():
                b2 = (b + 2) % _NB
                pltpu.make_async_copy(
                    sd_hbm.at[0], idx.at[b2], isems[b2]).wait()
                pltpu.async_copy(
                    g_hbm.at[idx.at[b2, 0]], rows.at[rb], gsems[rb])
        return 0
    lax.fori_loop(0, _NCHUNK // _NB, grp, 0)

    plsc.subcore_barrier()
    r0 = s * _RPS
    for k in range(_RPS // 64):
        pltpu.sync_copy(acc_sh.at[pl.ds(r0 + k * 64, 64)], zbuf)
        pltpu.sync_copy(zbuf, out_hbm.at[c, pl.ds(r0 + k * 64, 64)])


@functools.cache
def _msg_call():
    return pl.kernel(
        _msg_body,
        out_type=jax.ShapeDtypeStruct((_NC, _NPAD, _H), jnp.float32),
        mesh=_mesh(),
        scratch_types=[
            pltpu.VMEM((_NB, 2, _CHUNK), jnp.int32),
            pltpu.VMEM((2, _CHUNK, _H), jnp.float32),
            pltpu.VMEM((64, _H), jnp.float32),
            pltpu.VMEM_SHARED((_NPAD, _H), jnp.float32),
        ] + [pltpu.SemaphoreType.DMA] * 6,
    )


def _rowspec(w):
    return pl.BlockSpec((_ROWBLK, w), lambda i: (i, 0))


def _fullspec(r, c):
    return pl.BlockSpec((r, c), lambda i: (0, 0))


def _dot(a, b):
    return jnp.dot(a, b, preferred_element_type=jnp.float32)


def _enc_body(x, w1, b1, w2, b2, o):
    h = jnp.maximum(_dot(x[...], w1[...]) + b1[...], 0.0)
    o[...] = _dot(h, w2[...]) + b2[...]


def _enc_call(x, w1, b1, w2, b2):
    return pl.pallas_call(
        _enc_body,
        grid=(_GRID,),
        in_specs=[_rowspec(_D), _fullspec(_D, _H), _fullspec(1, _H),
                  _fullspec(_H, _H), _fullspec(1, _H)],
        out_specs=_rowspec(_H),
        out_shape=jax.ShapeDtypeStruct((_NPAD, _H), jnp.float32),
    )(x, w1, b1, w2, b2)


def _pre_body(h, d0, d1, w, g, dinv):
    deg = d0[...] + d1[...] + 1.0
    dv = lax.rsqrt(deg)
    dv = dv * (1.5 - 0.5 * deg * dv * dv)
    dinv[...] = dv
    g[...] = dv * _dot(h[...], w[...])


def _pre_call(h, d0, d1, w):
    return pl.pallas_call(
        _pre_body,
        grid=(_GRID,),
        in_specs=[_rowspec(_H), _rowspec(1), _rowspec(1), _fullspec(_H, _H)],
        out_specs=(_rowspec(_H), _rowspec(1)),
        out_shape=(jax.ShapeDtypeStruct((_NPAD, _H), jnp.float32),
                   jax.ShapeDtypeStruct((_NPAD, 1), jnp.float32)),
    )(h, d0, d1, w)


def _mid_body(a0, a1, g, dinv, bp, wn, o):
    dv = dinv[...]
    h = jnp.maximum(dv * (a0[...] + a1[...] + g[...]) + bp[...], 0.0)
    o[...] = dv * _dot(h, wn[...])


def _mid_call(a0, a1, g, dinv, bp, wn):
    return pl.pallas_call(
        _mid_body,
        grid=(_GRID,),
        in_specs=[_rowspec(_H), _rowspec(_H), _rowspec(_H), _rowspec(1),
                  _fullspec(1, _H), _fullspec(_H, _H)],
        out_specs=_rowspec(_H),
        out_shape=jax.ShapeDtypeStruct((_NPAD, _H), jnp.float32),
    )(a0, a1, g, dinv, bp, wn)


def _dec_body(a0, a1, g, dinv, bc, w1, b1, w2, b2, o):
    h = jnp.maximum(dinv[...] * (a0[...] + a1[...] + g[...]) + bc[...], 0.0)
    t = jnp.maximum(_dot(h, w1[...]) + b1[...], 0.0)
    o[...] = jax.nn.sigmoid(_dot(t, w2[...]) + b2[...])


def _dec_call(a0, a1, g, dinv, bc, w1, b1, w2, b2):
    return pl.pallas_call(
        _dec_body,
        grid=(_GRID,),
        in_specs=[_rowspec(_H), _rowspec(_H), _rowspec(_H), _rowspec(1),
                  _fullspec(1, _H), _fullspec(_H, _H), _fullspec(1, _H),
                  _fullspec(_H, _D), _fullspec(1, _D)],
        out_specs=_rowspec(_D),
        out_shape=jax.ShapeDtypeStruct((_NPAD, _D), jnp.float32),
    )(a0, a1, g, dinv, bc, w1, b1, w2, b2)


def kernel(x, edge_index, W_e1, b_e1, W_e2, b_e2, W_c1, b_c1, W_c2, b_c2,
           W_c3, b_c3, W_d1, b_d1, W_d2, b_d2):
    src, dst = edge_index[0], edge_index[1]
    pad = _N + (jnp.arange(_EPAD - _E, dtype=jnp.int32) % (_NPAD - _N))
    srcp = jnp.concatenate([src, pad])
    dstp = jnp.concatenate([dst, pad])
    src2 = srcp.reshape(_EPAD // _CHUNK, _CHUNK)
    dst2 = dstp.reshape(_EPAD // _CHUNK, _CHUNK)
    sd = jnp.stack([src2, dst2], axis=1)
    xp = jnp.pad(x, ((0, _NPAD - _N), (0, 0)))

    degp = _deg_call()(dstp)
    d0 = degp[0].reshape(_NPAD, 1)
    d1 = degp[1].reshape(_NPAD, 1)

    h = _enc_call(xp, W_e1, b_e1.reshape(1, _H), W_e2, b_e2.reshape(1, _H))
    g1, dinv = _pre_call(h, d0, d1, W_c1)
    msg = _msg_call()
    acc = msg(g1, sd)
    g2 = _mid_call(acc[0], acc[1], g1, dinv, b_c1.reshape(1, _H), W_c2)
    acc = msg(g2, sd)
    g3 = _mid_call(acc[0], acc[1], g2, dinv, b_c2.reshape(1, _H), W_c3)
    acc = msg(g3, sd)
    out = _dec_call(acc[0], acc[1], g3, dinv, b_c3.reshape(1, _H),
                    W_d1, b_d1.reshape(1, _H), W_d2, b_d2.reshape(1, _D))
    return out[:_N]


# speedup vs baseline: 4.0241x; 1.0404x over previous
import functools

import jax
import jax.numpy as jnp
from jax import lax
from jax.experimental import pallas as pl
from jax.experimental.pallas import tpu as pltpu
from jax.experimental.pallas import tpu_sc as plsc

_N = 10000
_D = 128
_H = 128
_E = 320000

_NC = 2
_NS = 16
_NW = _NC * _NS

_CHUNK = 96
_NCHUNK = 108
_NB = 4
_NR = 3
_EPT = _CHUNK * _NCHUNK
_EPAD = _EPT * _NW
_NPAD = 10240
_RPT = _NPAD // _NW
_RPS = _NPAD // _NS

_ROWBLK = 512
_GRID = _NPAD // _ROWBLK

@functools.cache
def _mesh():
    return plsc.VectorSubcoreMesh(
        core_axis_name="c", subcore_axis_name="s",
        num_cores=_NC, num_subcores=_NS)


def _deg_body(dst_hbm, out_hbm, idx_d, ones_v, zrow, deg_sh, sem):
    c = lax.axis_index("c")
    s = lax.axis_index("s")
    wid = c * _NS + s

    one16 = jnp.ones((16,), jnp.float32)
    zero16 = jnp.zeros((16,), jnp.float32)
    for j in range(_CHUNK // 16):
        ones_v[pl.ds(j * 16, 16)] = one16

    def zfill(i, _):
        zrow[pl.ds(i * 16, 16)] = zero16
        return 0
    lax.fori_loop(0, _RPS // 16, zfill, 0)
    pltpu.sync_copy(zrow, deg_sh.at[pl.ds(s * _RPS, _RPS)])
    plsc.subcore_barrier()

    base = wid * _EPT

    def body(j, _):
        off = pl.multiple_of(base + j * _CHUNK, 8)
        pltpu.sync_copy(dst_hbm.at[pl.ds(off, _CHUNK)], idx_d)
        pltpu.sync_copy(ones_v, deg_sh.at[idx_d], add=True)
        return 0
    lax.fori_loop(0, _NCHUNK, body, 0)

    plsc.subcore_barrier()
    pltpu.sync_copy(deg_sh.at[pl.ds(s * _RPS, _RPS)], zrow)
    pltpu.sync_copy(zrow, out_hbm.at[c, pl.ds(s * _RPS, _RPS)])


@functools.cache
def _deg_call():
    return pl.kernel(
        _deg_body,
        out_type=jax.ShapeDtypeStruct((_NC, _NPAD), jnp.float32),
        mesh=_mesh(),
        scratch_types=[
            pltpu.VMEM((_CHUNK,), jnp.int32),
            pltpu.VMEM((_CHUNK,), jnp.float32),
            pltpu.VMEM((_RPS,), jnp.float32),
            pltpu.VMEM_SHARED((_NPAD,), jnp.float32),
            pltpu.SemaphoreType.DMA,
        ],
    )


def _msg_body(g_hbm, sd_hbm, out_hbm, idx, rows, zbuf, acc_sh,
              isem0, isem1, isem2, isem3, gsem0, gsem1, gsem2):
    c = lax.axis_index("c")
    s = lax.axis_index("s")
    wid = c * _NS + s
    isems = (isem0, isem1, isem2, isem3)
    gsems = (gsem0, gsem1, gsem2)

    c0 = wid * _NCHUNK
    for b in range(_NB):
        pltpu.async_copy(sd_hbm.at[c0 + b], idx.at[b], isems[b])

    zero16 = jnp.zeros((16,), jnp.float32)

    def zfill(i, _):
        for j in range(_H // 16):
            zbuf[i, pl.ds(j * 16, 16)] = zero16
        return 0
    lax.fori_loop(0, 64, zfill, 0)
    for k in range(_RPS // 64):
        pltpu.sync_copy(zbuf, acc_sh.at[pl.ds(s * _RPS + k * 64, 64)])
    plsc.subcore_barrier()

    for b in range(2):
        pltpu.make_async_copy(sd_hbm.at[0], idx.at[b], isems[b]).wait()
        pltpu.async_copy(g_hbm.at[idx.at[b, 0]], rows.at[b], gsems[b])

    def grp(jg, _):
        for u in range(12):
            j = jg * 12 + u
            bi = u % _NB
            b2 = (u + 2) % _NB
            r = u % _NR
            r2 = (u + 2) % _NR

            @pl.when(j + 2 < _NCHUNK)
            def _():
                pltpu.make_async_copy(
                    sd_hbm.at[0], idx.at[b2], isems[b2]).wait()
                pltpu.async_copy(
                    g_hbm.at[idx.at[b2, 0]], rows.at[r2], gsems[r2])

            pltpu.make_async_copy(
                g_hbm.at[pl.ds(0, _CHUNK)], rows.at[r], gsems[r]).wait()
            pltpu.sync_copy(rows.at[r], acc_sh.at[idx.at[bi, 1]], add=True)

            @pl.when(j + _NB < _NCHUNK)
            def _():
                pltpu.async_copy(sd_hbm.at[c0 + j + _NB], idx.at[bi], isems[bi])
        return 0
    lax.fori_loop(0, _NCHUNK // 12, grp, 0)

    plsc.subcore_barrier()
    r0 = s * _RPS
    for k in range(_RPS // 64):
        pltpu.sync_copy(acc_sh.at[pl.ds(r0 + k * 64, 64)], zbuf)
        pltpu.sync_copy(zbuf, out_hbm.at[c, pl.ds(r0 + k * 64, 64)])


@functools.cache
def _msg_call():
    return pl.kernel(
        _msg_body,
        out_type=jax.ShapeDtypeStruct((_NC, _NPAD, _H), jnp.float32),
        mesh=_mesh(),
        scratch_types=[
            pltpu.VMEM((_NB, 2, _CHUNK), jnp.int32),
            pltpu.VMEM((_NR, _CHUNK, _H), jnp.float32),
            pltpu.VMEM((64, _H), jnp.float32),
            pltpu.VMEM_SHARED((_NPAD, _H), jnp.float32),
        ] + [pltpu.SemaphoreType.DMA] * 7,
    )


def _rowspec(w):
    return pl.BlockSpec((_ROWBLK, w), lambda i: (i, 0))


def _fullspec(r, c):
    return pl.BlockSpec((r, c), lambda i: (0, 0))


def _dot(a, b):
    return jnp.dot(a, b, preferred_element_type=jnp.float32)


def _enc_body(x, w1, b1, w2, b2, o):
    h = jnp.maximum(_dot(x[...], w1[...]) + b1[...], 0.0)
    o[...] = _dot(h, w2[...]) + b2[...]


def _enc_call(x, w1, b1, w2, b2):
    return pl.pallas_call(
        _enc_body,
        grid=(_GRID,),
        in_specs=[_rowspec(_D), _fullspec(_D, _H), _fullspec(1, _H),
                  _fullspec(_H, _H), _fullspec(1, _H)],
        out_specs=_rowspec(_H),
        out_shape=jax.ShapeDtypeStruct((_NPAD, _H), jnp.float32),
    )(x, w1, b1, w2, b2)


def _pre_body(h, d0, d1, w, g, dinv):
    deg = d0[...] + d1[...] + 1.0
    dv = lax.rsqrt(deg)
    dv = dv * (1.5 - 0.5 * deg * dv * dv)
    dinv[...] = dv
    g[...] = dv * _dot(h[...], w[...])


def _pre_call(h, d0, d1, w):
    return pl.pallas_call(
        _pre_body,
        grid=(_GRID,),
        in_specs=[_rowspec(_H), _rowspec(1), _rowspec(1), _fullspec(_H, _H)],
        out_specs=(_rowspec(_H), _rowspec(1)),
        out_shape=(jax.ShapeDtypeStruct((_NPAD, _H), jnp.float32),
                   jax.ShapeDtypeStruct((_NPAD, 1), jnp.float32)),
    )(h, d0, d1, w)


def _mid_body(a0, a1, g, dinv, bp, wn, o):
    dv = dinv[...]
    h = jnp.maximum(dv * (a0[...] + a1[...] + g[...]) + bp[...], 0.0)
    o[...] = dv * _dot(h, wn[...])


def _mid_call(a0, a1, g, dinv, bp, wn):
    return pl.pallas_call(
        _mid_body,
        grid=(_GRID,),
        in_specs=[_rowspec(_H), _rowspec(_H), _rowspec(_H), _rowspec(1),
                  _fullspec(1, _H), _fullspec(_H, _H)],
        out_specs=_rowspec(_H),
        out_shape=jax.ShapeDtypeStruct((_NPAD, _H), jnp.float32),
    )(a0, a1, g, dinv, bp, wn)


def _dec_body(a0, a1, g, dinv, bc, w1, b1, w2, b2, o):
    h = jnp.maximum(dinv[...] * (a0[...] + a1[...] + g[...]) + bc[...], 0.0)
    t = jnp.maximum(_dot(h, w1[...]) + b1[...], 0.0)
    o[...] = jax.nn.sigmoid(_dot(t, w2[...]) + b2[...])


def _dec_call(a0, a1, g, dinv, bc, w1, b1, w2, b2):
    return pl.pallas_call(
        _dec_body,
        grid=(_GRID,),
        in_specs=[_rowspec(_H), _rowspec(_H), _rowspec(_H), _rowspec(1),
                  _fullspec(1, _H), _fullspec(_H, _H), _fullspec(1, _H),
                  _fullspec(_H, _D), _fullspec(1, _D)],
        out_specs=_rowspec(_D),
        out_shape=jax.ShapeDtypeStruct((_NPAD, _D), jnp.float32),
    )(a0, a1, g, dinv, bc, w1, b1, w2, b2)


def kernel(x, edge_index, W_e1, b_e1, W_e2, b_e2, W_c1, b_c1, W_c2, b_c2,
           W_c3, b_c3, W_d1, b_d1, W_d2, b_d2):
    src, dst = edge_index[0], edge_index[1]
    pad = _N + (jnp.arange(_EPAD - _E, dtype=jnp.int32) % (_NPAD - _N))
    srcp = jnp.concatenate([src, pad])
    dstp = jnp.concatenate([dst, pad])
    src2 = srcp.reshape(_EPAD // _CHUNK, _CHUNK)
    dst2 = dstp.reshape(_EPAD // _CHUNK, _CHUNK)
    sd = jnp.stack([src2, dst2], axis=1)
    xp = jnp.pad(x, ((0, _NPAD - _N), (0, 0)))

    degp = _deg_call()(dstp)
    d0 = degp[0].reshape(_NPAD, 1)
    d1 = degp[1].reshape(_NPAD, 1)

    h = _enc_call(xp, W_e1, b_e1.reshape(1, _H), W_e2, b_e2.reshape(1, _H))
    g1, dinv = _pre_call(h, d0, d1, W_c1)
    msg = _msg_call()
    acc = msg(g1, sd)
    g2 = _mid_call(acc[0], acc[1], g1, dinv, b_c1.reshape(1, _H), W_c2)
    acc = msg(g2, sd)
    g3 = _mid_call(acc[0], acc[1], g2, dinv, b_c2.reshape(1, _H), W_c3)
    acc = msg(g3, sd)
    out = _dec_call(acc[0], acc[1], g3, dinv, b_c3.reshape(1, _H),
                    W_d1, b_d1.reshape(1, _H), W_d2, b_d2.reshape(1, _D))
    return out[:_N]


# speedup vs baseline: 4.2476x; 1.0556x over previous
import functools

import jax
import jax.numpy as jnp
from jax import lax
from jax.experimental import pallas as pl
from jax.experimental.pallas import tpu as pltpu
from jax.experimental.pallas import tpu_sc as plsc

_N = 10000
_D = 128
_H = 128
_E = 320000

_NC = 2
_NS = 16
_NW = _NC * _NS

_CHUNK = 96
_NCHUNK = 108
_NB = 4
_NR = 3
_EPT = _CHUNK * _NCHUNK
_EPAD = _EPT * _NW
_NPAD = 10240
_RPT = _NPAD // _NW
_RPS = _NPAD // _NS

_ROWBLK = 512
_GRID = _NPAD // _ROWBLK

@functools.cache
def _mesh():
    return plsc.VectorSubcoreMesh(
        core_axis_name="c", subcore_axis_name="s",
        num_cores=_NC, num_subcores=_NS)


def _deg_body(dst_hbm, out_hbm, idx_d, idx_d2, ones_v, zrow, deg_sh, sem,
              sem2):
    c = lax.axis_index("c")
    s = lax.axis_index("s")
    wid = c * _NS + s

    one16 = jnp.ones((16,), jnp.float32)
    zero16 = jnp.zeros((16,), jnp.float32)
    for j in range(_CHUNK // 16):
        ones_v[pl.ds(j * 16, 16)] = one16

    def zfill(i, _):
        zrow[pl.ds(i * 16, 16)] = zero16
        return 0
    lax.fori_loop(0, _RPS // 16, zfill, 0)
    pltpu.sync_copy(zrow, deg_sh.at[pl.ds(s * _RPS, _RPS)])
    plsc.subcore_barrier()

    base = wid * _EPT

    def off(j):
        return pl.multiple_of(base + j * _CHUNK, 8)

    pltpu.async_copy(dst_hbm.at[pl.ds(off(0), _CHUNK)], idx_d, sem)
    pltpu.async_copy(dst_hbm.at[pl.ds(off(1), _CHUNK)], idx_d2, sem2)
    sems = (sem, sem2)
    bufs = (idx_d, idx_d2)

    def body(jg, _):
        for b in range(2):
            j = jg * 2 + b
            pltpu.make_async_copy(
                dst_hbm.at[pl.ds(0, _CHUNK)], bufs[b], sems[b]).wait()
            pltpu.sync_copy(ones_v, deg_sh.at[bufs[b]], add=True)

            @pl.when(j + 2 < _NCHUNK)
            def _():
                pltpu.async_copy(
                    dst_hbm.at[pl.ds(off(j + 2), _CHUNK)], bufs[b], sems[b])
        return 0
    lax.fori_loop(0, _NCHUNK // 2, body, 0)

    plsc.subcore_barrier()
    pltpu.sync_copy(deg_sh.at[pl.ds(s * _RPS, _RPS)], zrow)
    pltpu.sync_copy(zrow, out_hbm.at[c, pl.ds(s * _RPS, _RPS)])


@functools.cache
def _deg_call():
    return pl.kernel(
        _deg_body,
        out_type=jax.ShapeDtypeStruct((_NC, _NPAD), jnp.float32),
        mesh=_mesh(),
        scratch_types=[
            pltpu.VMEM((_CHUNK,), jnp.int32),
            pltpu.VMEM((_CHUNK,), jnp.int32),
            pltpu.VMEM((_CHUNK,), jnp.float32),
            pltpu.VMEM((_RPS,), jnp.float32),
            pltpu.VMEM_SHARED((_NPAD,), jnp.float32),
            pltpu.SemaphoreType.DMA,
            pltpu.SemaphoreType.DMA,
        ],
    )


def _msg_body(g_hbm, sd_hbm, out_hbm, idx, rows, zbuf, acc_sh,
              isem0, isem1, isem2, isem3, gsem0, gsem1, gsem2):
    c = lax.axis_index("c")
    s = lax.axis_index("s")
    wid = c * _NS + s
    isems = (isem0, isem1, isem2, isem3)
    gsems = (gsem0, gsem1, gsem2)

    c0 = wid * _NCHUNK
    for b in range(_NB):
        pltpu.async_copy(sd_hbm.at[c0 + b], idx.at[b], isems[b])

    zero16 = jnp.zeros((16,), jnp.float32)

    def zfill(i, _):
        for j in range(_H // 16):
            zbuf[i, pl.ds(j * 16, 16)] = zero16
        return 0
    lax.fori_loop(0, 64, zfill, 0)
    for k in range(_RPS // 64):
        pltpu.sync_copy(zbuf, acc_sh.at[pl.ds(s * _RPS + k * 64, 64)])
    plsc.subcore_barrier()

    for b in range(2):
        pltpu.make_async_copy(sd_hbm.at[0], idx.at[b], isems[b]).wait()
        pltpu.async_copy(g_hbm.at[idx.at[b, 0]], rows.at[b], gsems[b])

    def grp(jg, _):
        for u in range(12):
            j = jg * 12 + u
            bi = u % _NB
            b2 = (u + 2) % _NB
            r = u % _NR
            r2 = (u + 2) % _NR

            @pl.when(j + 2 < _NCHUNK)
            def _():
                pltpu.make_async_copy(
                    sd_hbm.at[0], idx.at[b2], isems[b2]).wait()
                pltpu.async_copy(
                    g_hbm.at[idx.at[b2, 0]], rows.at[r2], gsems[r2])

            pltpu.make_async_copy(
                g_hbm.at[pl.ds(0, _CHUNK)], rows.at[r], gsems[r]).wait()
            pltpu.sync_copy(rows.at[r], acc_sh.at[idx.at[bi, 1]], add=True)

            @pl.when(j + _NB < _NCHUNK)
            def _():
                pltpu.async_copy(sd_hbm.at[c0 + j + _NB], idx.at[bi], isems[bi])
        return 0
    lax.fori_loop(0, _NCHUNK // 12, grp, 0)

    plsc.subcore_barrier()
    r0 = s * _RPS
    for k in range(_RPS // 64):
        pltpu.sync_copy(acc_sh.at[pl.ds(r0 + k * 64, 64)], zbuf)
        pltpu.sync_copy(zbuf, out_hbm.at[c, pl.ds(r0 + k * 64, 64)])


@functools.cache
def _msg_call():
    return pl.kernel(
        _msg_body,
        out_type=jax.ShapeDtypeStruct((_NC, _NPAD, _H), jnp.float32),
        mesh=_mesh(),
        scratch_types=[
            pltpu.VMEM((_NB, 2, _CHUNK), jnp.int32),
            pltpu.VMEM((_NR, _CHUNK, _H), jnp.float32),
            pltpu.VMEM((64, _H), jnp.float32),
            pltpu.VMEM_SHARED((_NPAD, _H), jnp.float32),
        ] + [pltpu.SemaphoreType.DMA] * 7,
    )


def _rowspec(w):
    return pl.BlockSpec((_ROWBLK, w), lambda i: (i, 0))


def _fullspec(r, c):
    return pl.BlockSpec((r, c), lambda i: (0, 0))


def _dot(a, b):
    return jnp.dot(a, b, preferred_element_type=jnp.float32)


def _encpre_body(x, w1, b1, w2, b2, d0, d1, wc, g, dinv):
    h = jnp.maximum(_dot(x[...], w1[...]) + b1[...], 0.0)
    h = _dot(h, w2[...]) + b2[...]
    deg = d0[...] + d1[...] + 1.0
    dv = lax.rsqrt(deg)
    dv = dv * (1.5 - 0.5 * deg * dv * dv)
    dinv[...] = dv
    g[...] = dv * _dot(h, wc[...])


def _encpre_call(x, w1, b1, w2, b2, d0, d1, wc):
    return pl.pallas_call(
        _encpre_body,
        grid=(_GRID,),
        in_specs=[_rowspec(_D), _fullspec(_D, _H), _fullspec(1, _H),
                  _fullspec(_H, _H), _fullspec(1, _H),
                  _rowspec(1), _rowspec(1), _fullspec(_H, _H)],
        out_specs=(_rowspec(_H), _rowspec(1)),
        out_shape=(jax.ShapeDtypeStruct((_NPAD, _H), jnp.float32),
                   jax.ShapeDtypeStruct((_NPAD, 1), jnp.float32)),
    )(x, w1, b1, w2, b2, d0, d1, wc)


def _mid_body(a0, a1, g, dinv, bp, wn, o):
    dv = dinv[...]
    h = jnp.maximum(dv * (a0[...] + a1[...] + g[...]) + bp[...], 0.0)
    o[...] = dv * _dot(h, wn[...])


def _mid_call(a0, a1, g, dinv, bp, wn):
    return pl.pallas_call(
        _mid_body,
        grid=(_GRID,),
        in_specs=[_rowspec(_H), _rowspec(_H), _rowspec(_H), _rowspec(1),
                  _fullspec(1, _H), _fullspec(_H, _H)],
        out_specs=_rowspec(_H),
        out_shape=jax.ShapeDtypeStruct((_NPAD, _H), jnp.float32),
    )(a0, a1, g, dinv, bp, wn)


def _dec_body(a0, a1, g, dinv, bc, w1, b1, w2, b2, o):
    h = jnp.maximum(dinv[...] * (a0[...] + a1[...] + g[...]) + bc[...], 0.0)
    t = jnp.maximum(_dot(h, w1[...]) + b1[...], 0.0)
    o[...] = jax.nn.sigmoid(_dot(t, w2[...]) + b2[...])


def _dec_call(a0, a1, g, dinv, bc, w1, b1, w2, b2):
    return pl.pallas_call(
        _dec_body,
        grid=(_GRID,),
        in_specs=[_rowspec(_H), _rowspec(_H), _rowspec(_H), _rowspec(1),
                  _fullspec(1, _H), _fullspec(_H, _H), _fullspec(1, _H),
                  _fullspec(_H, _D), _fullspec(1, _D)],
        out_specs=_rowspec(_D),
        out_shape=jax.ShapeDtypeStruct((_NPAD, _D), jnp.float32),
    )(a0, a1, g, dinv, bc, w1, b1, w2, b2)


def kernel(x, edge_index, W_e1, b_e1, W_e2, b_e2, W_c1, b_c1, W_c2, b_c2,
           W_c3, b_c3, W_d1, b_d1, W_d2, b_d2):
    src, dst = edge_index[0], edge_index[1]
    pad = _N + (jnp.arange(_EPAD - _E, dtype=jnp.int32) % (_NPAD - _N))
    srcp = jnp.concatenate([src, pad])
    dstp = jnp.concatenate([dst, pad])
    src2 = srcp.reshape(_EPAD // _CHUNK, _CHUNK)
    dst2 = dstp.reshape(_EPAD // _CHUNK, _CHUNK)
    sd = jnp.stack([src2, dst2], axis=1)
    xp = jnp.pad(x, ((0, _NPAD - _N), (0, 0)))

    degp = _deg_call()(dstp)
    d0 = degp[0].reshape(_NPAD, 1)
    d1 = degp[1].reshape(_NPAD, 1)

    g1, dinv = _encpre_call(xp, W_e1, b_e1.reshape(1, _H), W_e2,
                            b_e2.reshape(1, _H), d0, d1, W_c1)
    msg = _msg_call()
    acc = msg(g1, sd)
    g2 = _mid_call(acc[0], acc[1], g1, dinv, b_c1.reshape(1, _H), W_c2)
    acc = msg(g2, sd)
    g3 = _mid_call(acc[0], acc[1], g2, dinv, b_c2.reshape(1, _H), W_c3)
    acc = msg(g3, sd)
    out = _dec_call(acc[0], acc[1], g3, dinv, b_c3.reshape(1, _H),
                    W_d1, b_d1.reshape(1, _H), W_d2, b_d2.reshape(1, _D))
    return out[:_N]
